# baseline (speedup 1.0000x reference)
import jax
import jax.numpy as jnp
import numpy as np
from jax import lax
from jax.experimental import pallas as pl
from jax.experimental.pallas import tpu as pltpu

_D_MODEL = 768
_N_EXPERTS = 8
_TOKENS = 32768
_BLOCK = 2048
_NBLK = _TOKENS // _BLOCK
_HW = _BLOCK // 2

_I = np.int32
_SIGN2 = _I(-2147450880)
_MAG2 = _I(0x7FFF7FFF)
_LSB2 = _I(0x00010001)
_RND2 = _I(0x00030003)
_EM2 = _I(0x0FFF0FFF)
_BIAS2 = _I(0x38003800)


def _f16x2_to_bf16x2(xi):
    lsb = lax.bitwise_and(lax.shift_right_logical(xi, _I(3)), _LSB2)
    mag = lax.bitwise_and(xi, _MAG2)
    r = mag + lsb + _RND2
    em = lax.bitwise_and(lax.shift_right_logical(r, _I(3)), _EM2) + _BIAS2
    return lax.bitwise_or(em, lax.bitwise_and(xi, _SIGN2))


def _monotone(b):
    mask = lax.bitwise_and(lax.shift_right_arithmetic(b, _I(31)), _I(0x7FFFFFFF))
    return lax.bitwise_xor(b, mask)


def _smax(v):
    for sh in (1, 2, 4):
        v = jnp.maximum(v, pltpu.roll(v, sh, 0))
    return v


def _router_block(x_any, w_ref, idx_ref, wts_ref, xbuf, sem):
    i = pl.program_id(0)
    xi32 = x_any.bitcast(jnp.int32)

    @pl.when(i == 0)
    def _():
        pltpu.make_async_copy(
            xi32.at[pl.ds(0, _HW)], xbuf.at[0], sem.at[0]).start()

    @pl.when(i + 1 < _NBLK)
    def _():
        nxt = (i + 1) % 2
        pltpu.make_async_copy(
            xi32.at[pl.ds((i + 1) * _HW, _HW)], xbuf.at[nxt], sem.at[nxt]).start()

    slot = i % 2
    pltpu.make_async_copy(
        xi32.at[pl.ds(i * _HW, _HW)], xbuf.at[slot], sem.at[slot]).wait()

    xi = xbuf[slot]
    xb = pltpu.bitcast(_f16x2_to_bf16x2(xi), jnp.bfloat16)
    w = w_ref[...].astype(jnp.bfloat16)
    scores = lax.dot_general(
        w, xb, dimension_numbers=(((1,), (1,)), ((), ())),
        preferred_element_type=jnp.float32,
    )

    sb = lax.bitcast_convert_type(scores, jnp.int32)
    rev_e = _I(7) - lax.broadcasted_iota(jnp.int32, scores.shape, 0)
    key = lax.bitwise_or(
        lax.bitwise_and(_monotone(sb), _I(-8)), rev_e)
    k1 = _smax(key)
    k2 = _smax(jnp.where(key == k1, _I(-2147483648), key))

    e1 = _I(7) - lax.bitwise_and(k1, _I(7))
    e2 = _I(7) - lax.bitwise_and(k2, _I(7))
    v1 = lax.bitcast_convert_type(
        _monotone(lax.bitwise_and(k1, _I(-8))), jnp.float32)
    v2 = lax.bitcast_convert_type(
        _monotone(lax.bitwise_and(k2, _I(-8))), jnp.float32)
    t = jnp.exp(v2 - v1)
    w1 = 1.0 / (1.0 + t)
    w2 = t / (1.0 + t)

    idx_ref[0:1, :] = e1[0:1, :]
    idx_ref[1:2, :] = e2[0:1, :]
    wts_ref[0:1, :] = w1[0:1, :]
    wts_ref[1:2, :] = w2[0:1, :]


def kernel(x, W):
    grid = (_NBLK,)
    idx_t, wts_t = pl.pallas_call(
        _router_block,
        grid=grid,
        in_specs=[
            pl.BlockSpec(memory_space=pl.ANY),
            pl.BlockSpec((_N_EXPERTS, _D_MODEL), lambda i: (0, 0)),
        ],
        out_specs=[
            pl.BlockSpec((2, _BLOCK), lambda i: (0, i)),
            pl.BlockSpec((2, _BLOCK), lambda i: (0, i)),
        ],
        out_shape=[
            jax.ShapeDtypeStruct((2, _TOKENS), jnp.int32),
            jax.ShapeDtypeStruct((2, _TOKENS), jnp.float32),
        ],
        scratch_shapes=[
            pltpu.VMEM((2, _HW, _D_MODEL), jnp.int32),
            pltpu.SemaphoreType.DMA((2,)),
        ],
        compiler_params=pltpu.CompilerParams(
            dimension_semantics=("arbitrary",),
        ),
    )(lax.bitcast_convert_type(x, jnp.bfloat16), W.astype(jnp.float32))
    return idx_t.T, wts_t.T


# speedup vs baseline: 1.0792x; 1.0792x over previous
import jax
import jax.numpy as jnp
import numpy as np
from jax import lax
from jax.experimental import pallas as pl
from jax.experimental.pallas import tpu as pltpu

_D_MODEL = 768
_N_EXPERTS = 8
_TOKENS = 32768
_BLOCK = 2048
_NBLK = _TOKENS // _BLOCK
_HW = _BLOCK // 2

_I = np.int32
_SIGN2 = _I(-2147450880)
_MAG2 = _I(0x7FFF7FFF)
_LSB2 = _I(0x00010001)
_RND2 = _I(0x00030003)
_EM2 = _I(0x0FFF0FFF)
_BIAS2 = _I(0x38003800)


def _f16x2_to_bf16x2(xi):
    lsb = lax.bitwise_and(lax.shift_right_logical(xi, _I(3)), _LSB2)
    mag = lax.bitwise_and(xi, _MAG2)
    r = mag + lsb + _RND2
    em = lax.bitwise_and(lax.shift_right_logical(r, _I(3)), _EM2) + _BIAS2
    return lax.bitwise_or(em, lax.bitwise_and(xi, _SIGN2))


def _monotone(b):
    mask = lax.bitwise_and(lax.shift_right_arithmetic(b, _I(31)), _I(0x7FFFFFFF))
    return lax.bitwise_xor(b, mask)


def _smax(v):
    for sh in (1, 2, 4):
        v = jnp.maximum(v, pltpu.roll(v, sh, 0))
    return v


_NBUF = 4


def _router_block(x_any, w_ref, idx_ref, wts_ref, xbuf, sem):
    i = pl.program_id(0)
    xi32 = x_any.bitcast(jnp.int32)

    @pl.when(i == 0)
    def _():
        for b in range(_NBUF - 1):
            pltpu.make_async_copy(
                xi32.at[pl.ds(b * _HW, _HW)], xbuf.at[b], sem.at[b]).start()

    @pl.when(i + _NBUF - 1 < _NBLK)
    def _():
        nxt = (i + _NBUF - 1) % _NBUF
        pltpu.make_async_copy(
            xi32.at[pl.ds((i + _NBUF - 1) * _HW, _HW)],
            xbuf.at[nxt], sem.at[nxt]).start()

    slot = i % _NBUF
    pltpu.make_async_copy(
        xi32.at[pl.ds(i * _HW, _HW)], xbuf.at[slot], sem.at[slot]).wait()

    xi = xbuf[slot]
    xb = pltpu.bitcast(_f16x2_to_bf16x2(xi), jnp.bfloat16)
    w = w_ref[...].astype(jnp.bfloat16)
    scores = lax.dot_general(
        w, xb, dimension_numbers=(((1,), (1,)), ((), ())),
        preferred_element_type=jnp.float32,
    )

    sb = lax.bitcast_convert_type(scores, jnp.int32)
    rev_e = _I(7) - lax.broadcasted_iota(jnp.int32, scores.shape, 0)
    key = lax.bitwise_or(
        lax.bitwise_and(_monotone(sb), _I(-8)), rev_e)
    k1 = _smax(key)
    k2 = _smax(jnp.where(key == k1, _I(-2147483648), key))

    e1 = _I(7) - lax.bitwise_and(k1, _I(7))
    e2 = _I(7) - lax.bitwise_and(k2, _I(7))
    v1 = lax.bitcast_convert_type(
        _monotone(lax.bitwise_and(k1, _I(-8))), jnp.float32)
    v2 = lax.bitcast_convert_type(
        _monotone(lax.bitwise_and(k2, _I(-8))), jnp.float32)
    t = jnp.exp(v2 - v1)
    w1 = 1.0 / (1.0 + t)
    w2 = t / (1.0 + t)

    idx_ref[0:1, :] = e1[0:1, :]
    idx_ref[1:2, :] = e2[0:1, :]
    wts_ref[0:1, :] = w1[0:1, :]
    wts_ref[1:2, :] = w2[0:1, :]


def _pallas_router(xb, W32):
    grid = (_NBLK,)
    idx_t, wts_t = pl.pallas_call(
        _router_block,
        grid=grid,
        in_specs=[
            pl.BlockSpec(memory_space=pl.ANY),
            pl.BlockSpec((_N_EXPERTS, _D_MODEL), lambda i: (0, 0)),
        ],
        out_specs=[
            pl.BlockSpec((2, _BLOCK), lambda i: (0, i)),
            pl.BlockSpec((2, _BLOCK), lambda i: (0, i)),
        ],
        out_shape=[
            jax.ShapeDtypeStruct((2, _TOKENS), jnp.int32),
            jax.ShapeDtypeStruct((2, _TOKENS), jnp.float32),
        ],
        scratch_shapes=[
            pltpu.VMEM((_NBUF, _HW, _D_MODEL), jnp.int32),
            pltpu.SemaphoreType.DMA((_NBUF,)),
        ],
        compiler_params=pltpu.CompilerParams(
            dimension_semantics=("arbitrary",),
        ),
    )(xb, W32)
    return idx_t.T, wts_t.T


def kernel(x, W):
    return _pallas_router(
        lax.bitcast_convert_type(x, jnp.bfloat16), W.astype(jnp.float32))
